# Initial kernel scaffold; baseline (speedup 1.0000x reference)
import jax
import jax.numpy as jnp
from jax.experimental import pallas as pl


def kernel(fit_X_col, donors_idx):
    raise NotImplementedError("write your pallas kernel here")


# speedup vs baseline: 1.3921x; 1.3921x over previous
import functools

import jax
import jax.numpy as jnp
from jax import lax
from jax.experimental import pallas as pl
from jax.experimental.pallas import tpu as pltpu
from jax.experimental.pallas import tpu_sc as plsc

_NUM_WORKERS = 32


def _make_gather(n_table, n_idx):
    per_w = n_idx // _NUM_WORKERS
    assert per_w * _NUM_WORKERS == n_idx and per_w % 8 == 0

    mesh = plsc.VectorSubcoreMesh(core_axis_name="c", subcore_axis_name="s")

    @functools.partial(
        pl.kernel,
        out_type=jax.ShapeDtypeStruct((n_idx,), jnp.float32),
        mesh=mesh,
        scratch_types=[
            pltpu.VMEM((per_w,), jnp.int32),
            pltpu.VMEM((per_w,), jnp.float32),
            pltpu.SemaphoreType.DMA,
        ],
    )
    def gather_kernel(table_hbm, idx_hbm, out_hbm, idx_v, val_v, sem):
        wid = lax.axis_index("s") * 2 + lax.axis_index("c")
        base = wid * per_w
        pltpu.sync_copy(idx_hbm.at[pl.ds(base, per_w)], idx_v)
        pltpu.async_copy(table_hbm.at[idx_v], val_v, sem).wait()
        pltpu.sync_copy(val_v, out_hbm.at[pl.ds(base, per_w)])

    return gather_kernel


def kernel(fit_X_col, donors_idx):
    idx_flat = donors_idx.astype(jnp.int32).ravel()
    out_flat = _make_gather(fit_X_col.shape[0], idx_flat.shape[0])(
        fit_X_col, idx_flat
    )
    return out_flat.reshape(donors_idx.shape)


# speedup vs baseline: 1.8071x; 1.2982x over previous
import functools

import jax
import jax.numpy as jnp
from jax import lax
from jax.experimental import pallas as pl
from jax.experimental.pallas import tpu as pltpu
from jax.experimental.pallas import tpu_sc as plsc

_NUM_WORKERS = 32


def _make_gather(idx_shape):
    n_rows, n_cols = idx_shape
    rows_w = n_rows // _NUM_WORKERS
    assert rows_w * _NUM_WORKERS == n_rows

    mesh = plsc.VectorSubcoreMesh(core_axis_name="c", subcore_axis_name="s")

    @functools.partial(
        pl.kernel,
        out_type=jax.ShapeDtypeStruct(idx_shape, jnp.float32),
        mesh=mesh,
        scratch_types=[
            pltpu.VMEM((rows_w, n_cols), jnp.int32),
            pltpu.VMEM((rows_w, n_cols), jnp.float32),
            pltpu.SemaphoreType.DMA,
        ],
    )
    def gather_kernel(table_hbm, idx_hbm, out_hbm, idx_v, val_v, sem):
        wid = lax.axis_index("s") * 2 + lax.axis_index("c")
        base = wid * rows_w
        pltpu.sync_copy(idx_hbm.at[pl.ds(base, rows_w), :], idx_v)

        def fire(j, carry):
            pltpu.async_copy(table_hbm.at[idx_v.at[j]], val_v.at[j], sem)
            return carry

        lax.fori_loop(0, rows_w, fire, 0)

        def drain(j, carry):
            pltpu.make_async_copy(
                table_hbm.at[idx_v.at[j]], val_v.at[j], sem
            ).wait()
            return carry

        lax.fori_loop(0, rows_w, drain, 0)
        pltpu.sync_copy(val_v, out_hbm.at[pl.ds(base, rows_w), :])

    return gather_kernel


def kernel(fit_X_col, donors_idx):
    idx = donors_idx.astype(jnp.int32)
    return _make_gather(idx.shape)(fit_X_col, idx)


# speedup vs baseline: 1.8335x; 1.0146x over previous
import functools

import jax
import jax.numpy as jnp
from jax import lax
from jax.experimental import pallas as pl
from jax.experimental.pallas import tpu as pltpu
from jax.experimental.pallas import tpu_sc as plsc

_NUM_WORKERS = 32


def _make_gather(idx_shape):
    n_rows, n_cols = idx_shape
    rows_w = n_rows // _NUM_WORKERS
    assert rows_w * _NUM_WORKERS == n_rows

    mesh = plsc.VectorSubcoreMesh(core_axis_name="c", subcore_axis_name="s")

    @functools.partial(
        pl.kernel,
        out_type=jax.ShapeDtypeStruct(idx_shape, jnp.float32),
        mesh=mesh,
        scratch_types=[
            pltpu.VMEM((rows_w, n_cols), jnp.int32),
            pltpu.VMEM((rows_w, n_cols), jnp.float32),
            pltpu.SemaphoreType.DMA,
        ],
    )
    def gather_kernel(table_hbm, idx_hbm, out_hbm, idx_v, val_v, sem):
        wid = lax.axis_index("s") * 2 + lax.axis_index("c")
        base = wid * rows_w
        pltpu.sync_copy(idx_hbm.at[pl.ds(base, rows_w), :], idx_v)

        unroll = 8

        def fire(j, carry):
            for g in range(unroll):
                r = j * unroll + g
                pltpu.async_copy(table_hbm.at[idx_v.at[r]], val_v.at[r], sem)
            return carry

        lax.fori_loop(0, rows_w // unroll, fire, 0)

        def drain(j, carry):
            for g in range(unroll):
                r = j * unroll + g
                pltpu.make_async_copy(
                    table_hbm.at[idx_v.at[r]], val_v.at[r], sem
                ).wait()
            return carry

        lax.fori_loop(0, rows_w // unroll, drain, 0)
        pltpu.sync_copy(val_v, out_hbm.at[pl.ds(base, rows_w), :])

    return gather_kernel


def kernel(fit_X_col, donors_idx):
    idx = donors_idx.astype(jnp.int32)
    return _make_gather(idx.shape)(fit_X_col, idx)
